# baseline (speedup 1.0000x reference)
import jax
import jax.numpy as jnp
from jax.experimental import pallas as pl
from jax.experimental.pallas import tpu as pltpu

N = 4096
D_IN = 128
D_OUT = 128
BM = 512
CHUNKS = N // BM
NBUF = 4


def _spmm_kernel(x_ref, w_ref, adj_hbm, out_ref, buf, xw_ref, sems):
    def start(chunk, slot):
        pltpu.make_async_copy(
            adj_hbm.at[pl.ds(chunk * BM, BM), :],
            buf.at[slot],
            sems.at[slot],
        ).start()

    for slot in range(NBUF):
        start(slot, slot)

    xw_ref[...] = jnp.dot(x_ref[...], w_ref[...],
                          preferred_element_type=jnp.float32)

    for chunk in range(CHUNKS):
        slot = chunk % NBUF
        pltpu.make_async_copy(
            adj_hbm.at[pl.ds(chunk * BM, BM), :],
            buf.at[slot],
            sems.at[slot],
        ).wait()
        mask = (buf[slot] != 0.0).astype(jnp.float32)
        out_ref[pl.ds(chunk * BM, BM), :] = jnp.dot(
            mask, xw_ref[...], preferred_element_type=jnp.float32)
        nxt = chunk + NBUF
        if nxt < CHUNKS:
            start(nxt, slot)


def kernel(input, adj, weight):
    return pl.pallas_call(
        _spmm_kernel,
        in_specs=[
            pl.BlockSpec(memory_space=pltpu.MemorySpace.VMEM),
            pl.BlockSpec(memory_space=pltpu.MemorySpace.VMEM),
            pl.BlockSpec(memory_space=pltpu.MemorySpace.HBM),
        ],
        out_specs=pl.BlockSpec(memory_space=pltpu.MemorySpace.VMEM),
        out_shape=jax.ShapeDtypeStruct((N, D_OUT), jnp.float32),
        scratch_shapes=[
            pltpu.VMEM((NBUF, BM, N), jnp.float32),
            pltpu.VMEM((N, D_OUT), jnp.float32),
            pltpu.SemaphoreType.DMA((NBUF,)),
        ],
    )(input, weight, adj)


# speedup vs baseline: 1.1311x; 1.1311x over previous
import jax
import jax.numpy as jnp
from jax.experimental import pallas as pl
from jax.experimental.pallas import tpu as pltpu

N = 4096
D_IN = 128
D_OUT = 128
BM = 512


def _spmm_kernel(x_ref, w_ref, adj_ref, out_ref, xw_ref):
    @pl.when(pl.program_id(0) == 0)
    def _():
        xw_ref[...] = jnp.dot(x_ref[...], w_ref[...],
                              preferred_element_type=jnp.float32).astype(
                                  jnp.bfloat16)

    mask = (adj_ref[...] != 0.0).astype(jnp.bfloat16)
    out_ref[...] = jnp.dot(mask, xw_ref[...],
                           preferred_element_type=jnp.float32)


def kernel(input, adj, weight):
    grid = (N // BM,)
    return pl.pallas_call(
        _spmm_kernel,
        grid=grid,
        in_specs=[
            pl.BlockSpec((N, D_IN), lambda i: (0, 0)),
            pl.BlockSpec((D_IN, D_OUT), lambda i: (0, 0)),
            pl.BlockSpec((BM, N), lambda i: (i, 0)),
        ],
        out_specs=pl.BlockSpec((BM, D_OUT), lambda i: (i, 0)),
        out_shape=jax.ShapeDtypeStruct((N, D_OUT), jnp.float32),
        scratch_shapes=[pltpu.VMEM((N, D_OUT), jnp.bfloat16)],
    )(input, weight, adj)


# speedup vs baseline: 1.1326x; 1.0014x over previous
import jax
import jax.numpy as jnp
from jax.experimental import pallas as pl
from jax.experimental.pallas import tpu as pltpu

N = 4096
D_IN = 128
D_OUT = 128
BM = 512


def _spmm_kernel(x_ref, w_ref, adj_ref, out_ref, xw_ref):
    @pl.when(pl.program_id(0) == 0)
    def _():
        xw_ref[...] = jnp.dot(x_ref[...], w_ref[...],
                              preferred_element_type=jnp.float32)

    mask = (adj_ref[...] != 0.0).astype(jnp.float32)
    out_ref[...] = jnp.dot(mask, xw_ref[...],
                           preferred_element_type=jnp.float32)


def kernel(input, adj, weight):
    grid = (N // BM,)
    return pl.pallas_call(
        _spmm_kernel,
        grid=grid,
        in_specs=[
            pl.BlockSpec((N, D_IN), lambda i: (0, 0)),
            pl.BlockSpec((D_IN, D_OUT), lambda i: (0, 0)),
            pl.BlockSpec((BM, N), lambda i: (i, 0)),
        ],
        out_specs=pl.BlockSpec((BM, D_OUT), lambda i: (i, 0)),
        out_shape=jax.ShapeDtypeStruct((N, D_OUT), jnp.float32),
        scratch_shapes=[pltpu.VMEM((N, D_OUT), jnp.float32)],
    )(input, weight, adj)


# speedup vs baseline: 1.1355x; 1.0025x over previous
import jax
import jax.numpy as jnp
from jax.experimental import pallas as pl

N = 4096
D_IN = 128
D_OUT = 128
BM = 512


def _spmm_kernel(x_ref, w_ref, adj_ref, out_ref):
    mask = (adj_ref[...] != 0.0).astype(jnp.float32)
    mx = jnp.dot(mask, x_ref[...], preferred_element_type=jnp.float32)
    out_ref[...] = jnp.dot(mx, w_ref[...],
                           preferred_element_type=jnp.float32)


def kernel(input, adj, weight):
    grid = (N // BM,)
    return pl.pallas_call(
        _spmm_kernel,
        grid=grid,
        in_specs=[
            pl.BlockSpec((N, D_IN), lambda i: (0, 0)),
            pl.BlockSpec((D_IN, D_OUT), lambda i: (0, 0)),
            pl.BlockSpec((BM, N), lambda i: (i, 0)),
        ],
        out_specs=pl.BlockSpec((BM, D_OUT), lambda i: (i, 0)),
        out_shape=jax.ShapeDtypeStruct((N, D_OUT), jnp.float32),
    )(input, weight, adj)
